# Initial kernel scaffold; baseline (speedup 1.0000x reference)
import jax
import jax.numpy as jnp
from jax.experimental import pallas as pl


def kernel(s_j, sbar_j, v_j, vbar_j, r_ij, nbrs, W1, b1, W2, b2, Wd, bd, Wdvbar, Wdv):
    raise NotImplementedError("write your pallas kernel here")


# speedup vs baseline: 13.8890x; 13.8890x over previous
import functools

import jax
import jax.numpy as jnp
from jax import lax
from jax.experimental import pallas as pl
from jax.experimental.pallas import tpu as pltpu
from jax.experimental.pallas import tpu_sc as plsc

FEAT = 128
NRBF = 20
CUTOFF = 5.0
N = 10000
E = 160000
H4 = 4 * FEAT
TW = H4 + 6 * FEAT
PW = 7 * FEAT
NPL = 13

NC = 2
NS = 16
NT = NC * NS
EPT = E // NT
GCH = 40
NGC = EPT // GCH

RPB = 128
NROW = E // RPB
RPC = NROW // NC
RPT = RPC // NS
ZR = 208

_HI = lax.Precision.HIGHEST


def _table_body(s_ref, vx, vy, vz, wx, wy, wz, W1_ref, b1_ref, W2_ref, b2_ref,
                out_ref):
    x = s_ref[...]
    pre = jnp.dot(x, W1_ref[...], preferred_element_type=jnp.float32,
                  precision=_HI) + b1_ref[...]
    act = pre * jax.nn.sigmoid(pre)
    h = jnp.dot(act, W2_ref[...], preferred_element_type=jnp.float32,
                precision=_HI) + b2_ref[...]
    out_ref[...] = jnp.concatenate(
        [h, vx[...], vy[...], vz[...], wx[...], wy[...], wz[...]], axis=1)


def _build_table(s_j, vx, vy, vz, wx, wy, wz, W1, b1, W2, b2):
    nb = 10
    bn = N // nb
    row = lambda i: (i, 0)
    full = lambda shape: pl.BlockSpec(shape, lambda i: (0, 0))
    return pl.pallas_call(
        _table_body,
        grid=(nb,),
        in_specs=[pl.BlockSpec((bn, FEAT), row)] * 7 + [
            full((FEAT, FEAT)), full((1, FEAT)),
            full((FEAT, H4)), full((1, H4))],
        out_specs=pl.BlockSpec((bn, TW), row),
        out_shape=jax.ShapeDtypeStruct((N, TW), jnp.float32),
    )(s_j, vx, vy, vz, wx, wy, wz, W1, b1.reshape(1, FEAT), W2,
      b2.reshape(1, H4))


def _gather_rows(table, i1r):
    mesh = plsc.VectorSubcoreMesh(core_axis_name="c", subcore_axis_name="s")

    @functools.partial(
        pl.kernel,
        out_type=jax.ShapeDtypeStruct((E, TW), jnp.float32),
        mesh=mesh,
        scratch_types=[
            pltpu.VMEM((NGC, GCH), jnp.int32),
            pltpu.VMEM((GCH, TW), jnp.float32),
            pltpu.SemaphoreType.DMA,
        ],
    )
    def k(t_hbm, idx_hbm, out_hbm, idx_v, buf, sem):
        w = lax.axis_index("c") * NS + lax.axis_index("s")
        pltpu.sync_copy(idx_hbm.at[w], idx_v)
        base = w * EPT

        @pl.loop(0, NGC)
        def _(kk):
            pltpu.async_copy(
                t_hbm.at[idx_v.at[kk]], buf, sem).wait()
            pltpu.sync_copy(buf, out_hbm.at[pl.ds(base + kk * GCH, GCH)])

    return k(table, i1r)


def _edge_body(g_ref, rx_ref, ry_ref, rz_ref, Wd_ref, bd_ref, p_ref):
    rx = rx_ref[...]
    ry = ry_ref[...]
    rz = rz_ref[...]
    d2 = rx * rx + ry * ry + rz * rz + 3e-8
    dist = jnp.sqrt(d2)
    inv = 1.0 / dist
    karr = (lax.broadcasted_iota(jnp.int32, (rx.shape[0], NRBF), 1)
            .astype(jnp.float32) + 1.0)
    ang = karr * ((jnp.pi / CUTOFF) * dist)
    rbf = jnp.sin(ang) * inv
    env = jnp.where(dist < CUTOFF,
                    0.5 * (jnp.cos(dist * (jnp.pi / CUTOFF)) + 1.0), 0.0)
    ws = (jnp.dot(rbf, Wd_ref[...], preferred_element_type=jnp.float32,
                  precision=_HI) + bd_ref[...]) * env
    g = g_ref[...]
    t0 = g[:, 0:FEAT] * ws[:, 0:FEAT]
    t1 = g[:, FEAT:2 * FEAT] * ws[:, FEAT:2 * FEAT]
    t2 = g[:, 2 * FEAT:3 * FEAT] * ws[:, 2 * FEAT:3 * FEAT]
    t3 = g[:, 3 * FEAT:4 * FEAT] * ws[:, 3 * FEAT:4 * FEAT]
    v1x = g[:, H4:H4 + FEAT]
    v1y = g[:, H4 + FEAT:H4 + 2 * FEAT]
    v1z = g[:, H4 + 2 * FEAT:H4 + 3 * FEAT]
    w1x = g[:, H4 + 3 * FEAT:H4 + 4 * FEAT]
    w1y = g[:, H4 + 4 * FEAT:H4 + 5 * FEAT]
    w1z = g[:, H4 + 5 * FEAT:H4 + 6 * FEAT]
    ux = rx * inv
    uy = ry * inv
    uz = rz * inv
    p_ref[...] = jnp.concatenate([
        t1,
        t2 * ux + t0 * v1x,
        t2 * uy + t0 * v1y,
        t2 * uz + t0 * v1z,
        t3 * w1x,
        t3 * w1y,
        t3 * w1z,
    ], axis=1)


def _edge_stage(G, rx, ry, rz, Wd, bd):
    eb = 640
    ng = E // eb
    row = lambda i: (i, 0)
    return pl.pallas_call(
        _edge_body,
        grid=(ng,),
        in_specs=[
            pl.BlockSpec((eb, TW), row),
            pl.BlockSpec((eb, 1), row),
            pl.BlockSpec((eb, 1), row),
            pl.BlockSpec((eb, 1), row),
            pl.BlockSpec((NRBF, H4), lambda i: (0, 0)),
            pl.BlockSpec((1, H4), lambda i: (0, 0)),
        ],
        out_specs=pl.BlockSpec((eb, PW), row),
        out_shape=jax.ShapeDtypeStruct((E, PW), jnp.float32),
    )(G, rx, ry, rz, Wd, bd.reshape(1, H4))


def _scatter_planes(P, G, i0r):
    mesh = plsc.VectorSubcoreMesh(core_axis_name="c", subcore_axis_name="s")

    @functools.partial(
        pl.kernel,
        out_type=jax.ShapeDtypeStruct((NC, NPL, N, FEAT), jnp.float32),
        mesh=mesh,
        scratch_types=[
            pltpu.VMEM((RPT + 1, 1, RPB), jnp.int32),
            pltpu.VMEM((RPB, FEAT), jnp.float32),
            pltpu.VMEM((ZR, FEAT), jnp.float32),
            pltpu.VMEM_SHARED((N, FEAT), jnp.float32),
            pltpu.SemaphoreType.DMA,
        ],
    )
    def k(p_hbm, g_hbm, idx_hbm, out_hbm, idx_v, pay_v, zero_v, acc, sem):
        c = lax.axis_index("c")
        w = lax.axis_index("s")

        @pl.loop(0, ZR)
        def _(i):
            @pl.loop(0, FEAT // 16)
            def _(l):
                zero_v.at[i, pl.ds(l * 16, 16)][...] = jnp.zeros(
                    (16,), jnp.float32)

        start = c * RPC + w * RPT
        pltpu.sync_copy(idx_hbm.at[pl.ds(start, RPT + 1)], idx_v)

        for p in range(NPL):
            for z in range(3):
                pltpu.sync_copy(zero_v,
                                acc.at[pl.ds(w * 624 + z * ZR, ZR)])

            @pl.when(w == NS - 1)
            def _():
                pltpu.sync_copy(zero_v.at[pl.ds(0, 16)],
                                acc.at[pl.ds(N - 16, 16)])

            plsc.subcore_barrier()

            if p < 7:
                src, col = p_hbm, p * FEAT
            else:
                src, col = g_hbm, H4 + (p - 7) * FEAT

            def body(j):
                rowg = (start + j) * RPB
                pltpu.sync_copy(
                    src.at[pl.ds(rowg, RPB), pl.ds(col, FEAT)], pay_v)
                pltpu.sync_copy(pay_v, acc.at[idx_v.at[j, 0]], add=True)

            @pl.loop(0, RPT)
            def _(j):
                body(j)

            @pl.when(w == NS - 1)
            def _():
                body(RPT)

            plsc.subcore_barrier()
            pltpu.sync_copy(acc.at[pl.ds(w * 624, 624)],
                            out_hbm.at[c, p, pl.ds(w * 624, 624)])

            @pl.when(w == NS - 1)
            def _():
                pltpu.sync_copy(acc.at[pl.ds(N - 16, 16)],
                                out_hbm.at[c, p, pl.ds(N - 16, 16)])

            plsc.subcore_barrier()

    return k(P, G, i0r)


def _node_body(a_ref, s_ref, sb_ref, vx_r, vy_r, vz_r, wx_r, wy_r, wz_r,
               Wvb_ref, Wv_ref,
               dh_ref, dhb_ref, dvx_r, dvy_r, dvz_r, dbx_r, dby_r, dbz_r):
    agg = lambda p: a_ref[p] + a_ref[NPL + p]
    dh = agg(0)
    pv = (agg(1), agg(2), agg(3))
    B = (agg(4), agg(5), agg(6))
    Av = (agg(7), agg(8), agg(9))
    Aw = (agg(10), agg(11), agg(12))
    s = s_ref[...]
    sb = sb_ref[...]
    v = (vx_r[...], vy_r[...], vz_r[...])
    vb = (wx_r[...], wy_r[...], wz_r[...])
    dh_ref[...] = dh
    dhb_ref[...] = v[0] * B[0] + v[1] * B[1] + v[2] * B[2]

    def cross(a, b, d):
        i, j = (d + 1) % 3, (d + 2) % 3
        return a[i] * b[j] - a[j] * b[i]

    Wvb = Wvb_ref[...]
    Wv = Wv_ref[...]
    douts = (dvx_r, dvy_r, dvz_r)
    bouts = (dbx_r, dby_r, dbz_r)
    for d in range(3):
        catb = jnp.concatenate(
            [s * Aw[d], sb * Av[d], cross(v, Av, d), cross(vb, Aw, d)], axis=1)
        bouts[d][...] = jnp.dot(catb, Wvb, preferred_element_type=jnp.float32,
                                precision=_HI)
        catv = jnp.concatenate(
            [s * Av[d], sb * Aw[d], cross(v, Aw, d)], axis=1)
        douts[d][...] = pv[d] + jnp.dot(
            catv, Wv, preferred_element_type=jnp.float32, precision=_HI)


def _node_stage(A, s_j, sbar_j, vx, vy, vz, wx, wy, wz, Wdvbar, Wdv):
    nb = 10
    bn = N // nb
    row = lambda i: (i, 0)
    o = pl.BlockSpec((bn, FEAT), row)
    outs = [jax.ShapeDtypeStruct((N, FEAT), jnp.float32)] * 8
    return pl.pallas_call(
        _node_body,
        grid=(nb,),
        in_specs=[
            pl.BlockSpec((2 * NPL, bn, FEAT), lambda i: (0, i, 0)),
        ] + [pl.BlockSpec((bn, FEAT), row)] * 8 + [
            pl.BlockSpec((H4, FEAT), lambda i: (0, 0)),
            pl.BlockSpec((3 * FEAT, FEAT), lambda i: (0, 0)),
        ],
        out_specs=[o] * 8,
        out_shape=outs,
    )(A, s_j, sbar_j, vx, vy, vz, wx, wy, wz, Wdvbar, Wdv)


def kernel(s_j, sbar_j, v_j, vbar_j, r_ij, nbrs, W1, b1, W2, b2, Wd, bd,
           Wdvbar, Wdv):
    i0 = nbrs[:, 0].astype(jnp.int32)
    i1 = nbrs[:, 1].astype(jnp.int32)
    vx, vy, vz = v_j[:, :, 0], v_j[:, :, 1], v_j[:, :, 2]
    wx, wy, wz = vbar_j[:, :, 0], vbar_j[:, :, 1], vbar_j[:, :, 2]
    rx, ry, rz = r_ij[:, 0:1], r_ij[:, 1:2], r_ij[:, 2:3]

    table = _build_table(s_j, vx, vy, vz, wx, wy, wz, W1, b1, W2, b2)
    G = _gather_rows(table, i1.reshape(NT, NGC, GCH))
    P = _edge_stage(G, rx, ry, rz, Wd, bd)
    parts = _scatter_planes(P, G, i0.reshape(NROW, 1, RPB))
    A = parts.reshape(NC * NPL, N, FEAT)
    dh, dhbar, dvx, dvy, dvz, dbx, dby, dbz = _node_stage(
        A, s_j, sbar_j, vx, vy, vz, wx, wy, wz, Wdvbar, Wdv)
    dv = jnp.stack([dvx, dvy, dvz], axis=-1)
    dvbar = jnp.stack([dbx, dby, dbz], axis=-1)
    return dh, dhbar, dv, dvbar


# speedup vs baseline: 19.7112x; 1.4192x over previous
import functools

import jax
import jax.numpy as jnp
from jax import lax
from jax.experimental import pallas as pl
from jax.experimental.pallas import tpu as pltpu
from jax.experimental.pallas import tpu_sc as plsc

FEAT = 128
NRBF = 20
CUTOFF = 5.0
N = 10000
E = 160000
H4 = 4 * FEAT
TW = H4 + 6 * FEAT
PW = 7 * FEAT
NPL = 13

NC = 2
NS = 16
NT = NC * NS
EPT = E // NT
GCH = 40
NGC = EPT // GCH

RPB = 128
NROW = E // RPB
RPC = NROW // NC
RPT = RPC // NS
ZR = 208

_HI = lax.Precision.HIGHEST


def _table_body(s_ref, vx, vy, vz, wx, wy, wz, W1_ref, b1_ref, W2_ref, b2_ref,
                out_ref):
    x = s_ref[...]
    pre = jnp.dot(x, W1_ref[...], preferred_element_type=jnp.float32,
                  precision=_HI) + b1_ref[...]
    act = pre * jax.nn.sigmoid(pre)
    h = jnp.dot(act, W2_ref[...], preferred_element_type=jnp.float32,
                precision=_HI) + b2_ref[...]
    out_ref[...] = jnp.concatenate(
        [h, vx[...], vy[...], vz[...], wx[...], wy[...], wz[...]], axis=1)


def _build_table(s_j, vx, vy, vz, wx, wy, wz, W1, b1, W2, b2):
    nb = 10
    bn = N // nb
    row = lambda i: (i, 0)
    full = lambda shape: pl.BlockSpec(shape, lambda i: (0, 0))
    return pl.pallas_call(
        _table_body,
        grid=(nb,),
        in_specs=[pl.BlockSpec((bn, FEAT), row)] * 7 + [
            full((FEAT, FEAT)), full((1, FEAT)),
            full((FEAT, H4)), full((1, H4))],
        out_specs=pl.BlockSpec((bn, TW), row),
        out_shape=jax.ShapeDtypeStruct((N, TW), jnp.float32),
    )(s_j, vx, vy, vz, wx, wy, wz, W1, b1.reshape(1, FEAT), W2,
      b2.reshape(1, H4))


def _gather_rows(table, i1r):
    mesh = plsc.VectorSubcoreMesh(core_axis_name="c", subcore_axis_name="s")

    @functools.partial(
        pl.kernel,
        out_type=jax.ShapeDtypeStruct((E, TW), jnp.float32),
        mesh=mesh,
        scratch_types=[
            pltpu.VMEM((NGC, GCH), jnp.int32),
            pltpu.VMEM((GCH, TW), jnp.float32),
            pltpu.VMEM((GCH, TW), jnp.float32),
            pltpu.SemaphoreType.DMA,
            pltpu.SemaphoreType.DMA,
        ],
    )
    def k(t_hbm, idx_hbm, out_hbm, idx_v, buf_a, buf_b, sem_a, sem_b):
        w = lax.axis_index("c") * NS + lax.axis_index("s")
        pltpu.sync_copy(idx_hbm.at[w], idx_v)
        base = w * EPT
        out_at = lambda kk: out_hbm.at[pl.ds(base + kk * GCH, GCH)]

        pltpu.async_copy(t_hbm.at[idx_v.at[0]], buf_a, sem_a)

        @pl.loop(0, NGC - 1, step=2)
        def _(kk):
            pltpu.async_copy(t_hbm.at[idx_v.at[kk + 1]], buf_b, sem_b)
            pltpu.make_async_copy(t_hbm.at[idx_v.at[kk]], buf_a, sem_a).wait()
            pltpu.sync_copy(buf_a, out_at(kk))
            pltpu.async_copy(t_hbm.at[idx_v.at[kk + 2]], buf_a, sem_a)
            pltpu.make_async_copy(
                t_hbm.at[idx_v.at[kk + 1]], buf_b, sem_b).wait()
            pltpu.sync_copy(buf_b, out_at(kk + 1))

        pltpu.make_async_copy(
            t_hbm.at[idx_v.at[NGC - 1]], buf_a, sem_a).wait()
        pltpu.sync_copy(buf_a, out_at(NGC - 1))

    return k(table, i1r)


def _rchain_body(rx_ref, ry_ref, rz_ref, out_ref):
    rx = rx_ref[...]
    ry = ry_ref[...]
    rz = rz_ref[...]
    d2 = rx * rx + ry * ry + rz * rz + 3e-8
    inv = lax.rsqrt(d2)
    dist = d2 * inv
    a = (jnp.pi / CUTOFF) * dist
    s1 = jnp.sin(a)
    c1 = jnp.cos(a)
    env = jnp.where(dist < CUTOFF, 0.5 * (c1 + 1.0), 0.0)
    m = env * inv
    two_c = 2.0 * c1
    rows = [m * s1]
    sk_prev, sk = s1, two_c * s1
    rows.append(m * sk)
    for _ in range(2, NRBF):
        sk_prev, sk = sk, two_c * sk - sk_prev
        rows.append(m * sk)
    rows.append(env)
    rows.append(rx * inv)
    rows.append(ry * inv)
    rows.append(rz * inv)
    out_ref[...] = jnp.concatenate(rows, axis=0)


def _rchain(rx, ry, rz):
    eb = 3200
    ng = E // eb
    col = lambda i: (0, i)
    return pl.pallas_call(
        _rchain_body,
        grid=(ng,),
        in_specs=[pl.BlockSpec((1, eb), col)] * 3,
        out_specs=pl.BlockSpec((NRBF + 4, eb), col),
        out_shape=jax.ShapeDtypeStruct((NRBF + 4, E), jnp.float32),
    )(rx, ry, rz)


def _edge_body(g_ref, rb_ref, Wd_ref, p_ref):
    rb = rb_ref[...]
    ws = jnp.dot(rb[:, 0:NRBF + 1], Wd_ref[...],
                 preferred_element_type=jnp.float32, precision=_HI)
    ux = rb[:, NRBF + 1:NRBF + 2]
    uy = rb[:, NRBF + 2:NRBF + 3]
    uz = rb[:, NRBF + 3:NRBF + 4]
    g = g_ref[...]
    t0 = g[:, 0:FEAT] * ws[:, 0:FEAT]
    t1 = g[:, FEAT:2 * FEAT] * ws[:, FEAT:2 * FEAT]
    t2 = g[:, 2 * FEAT:3 * FEAT] * ws[:, 2 * FEAT:3 * FEAT]
    t3 = g[:, 3 * FEAT:4 * FEAT] * ws[:, 3 * FEAT:4 * FEAT]
    v1x = g[:, H4:H4 + FEAT]
    v1y = g[:, H4 + FEAT:H4 + 2 * FEAT]
    v1z = g[:, H4 + 2 * FEAT:H4 + 3 * FEAT]
    w1x = g[:, H4 + 3 * FEAT:H4 + 4 * FEAT]
    w1y = g[:, H4 + 4 * FEAT:H4 + 5 * FEAT]
    w1z = g[:, H4 + 5 * FEAT:H4 + 6 * FEAT]
    p_ref[...] = jnp.concatenate([
        t1,
        t2 * ux + t0 * v1x,
        t2 * uy + t0 * v1y,
        t2 * uz + t0 * v1z,
        t3 * w1x,
        t3 * w1y,
        t3 * w1z,
    ], axis=1)


def _edge_stage(G, RB, Wd21):
    eb = 640
    ng = E // eb
    row = lambda i: (i, 0)
    return pl.pallas_call(
        _edge_body,
        grid=(ng,),
        in_specs=[
            pl.BlockSpec((eb, TW), row),
            pl.BlockSpec((eb, NRBF + 4), row),
            pl.BlockSpec((NRBF + 1, H4), lambda i: (0, 0)),
        ],
        out_specs=pl.BlockSpec((eb, PW), row),
        out_shape=jax.ShapeDtypeStruct((E, PW), jnp.float32),
    )(G, RB, Wd21)


def _scatter_planes(P, G, i0r):
    mesh = plsc.VectorSubcoreMesh(core_axis_name="c", subcore_axis_name="s")

    @functools.partial(
        pl.kernel,
        out_type=jax.ShapeDtypeStruct((NC, NPL, N, FEAT), jnp.float32),
        mesh=mesh,
        scratch_types=[
            pltpu.VMEM((RPT + 1, 1, RPB), jnp.int32),
            pltpu.VMEM((RPB, FEAT), jnp.float32),
            pltpu.VMEM((RPB, FEAT), jnp.float32),
            pltpu.VMEM_SHARED((N, FEAT), jnp.float32),
            pltpu.SemaphoreType.DMA,
            pltpu.SemaphoreType.DMA,
        ],
    )
    def k(p_hbm, g_hbm, idx_hbm, out_hbm, idx_v, pay_a, pay_b, acc,
          sem_a, sem_b):
        c = lax.axis_index("c")
        w = lax.axis_index("s")

        start = c * RPC + w * RPT
        pltpu.sync_copy(idx_hbm.at[pl.ds(start, RPT + 1)], idx_v)

        for p in range(NPL):
            @pl.loop(0, RPB)
            def _(i):
                @pl.loop(0, FEAT // 16)
                def _(l):
                    pay_a.at[i, pl.ds(l * 16, 16)][...] = jnp.zeros(
                        (16,), jnp.float32)

            for z in range(4):
                pltpu.sync_copy(pay_a,
                                acc.at[pl.ds(w * 624 + z * RPB, RPB)])
            pltpu.sync_copy(pay_a.at[pl.ds(0, 112)],
                            acc.at[pl.ds(w * 624 + 4 * RPB, 112)])

            @pl.when(w == NS - 1)
            def _():
                pltpu.sync_copy(pay_a.at[pl.ds(0, 16)],
                                acc.at[pl.ds(N - 16, 16)])

            plsc.subcore_barrier()

            if p < 7:
                src, col = p_hbm, p * FEAT
            else:
                src, col = g_hbm, H4 + (p - 7) * FEAT

            def src_at(j):
                return src.at[pl.ds((start + j) * RPB, RPB), pl.ds(col, FEAT)]

            def fetch(j, buf, sem):
                pltpu.async_copy(src_at(j), buf, sem)

            def drain_add(j, buf, sem):
                pltpu.make_async_copy(src_at(j), buf, sem).wait()
                pltpu.sync_copy(buf, acc.at[idx_v.at[j, 0]], add=True)

            fetch(0, pay_a, sem_a)

            @pl.loop(0, RPT - 1, step=2)
            def _(j):
                fetch(j + 1, pay_b, sem_b)
                drain_add(j, pay_a, sem_a)
                fetch(j + 2, pay_a, sem_a)
                drain_add(j + 1, pay_b, sem_b)

            drain_add(RPT - 1, pay_a, sem_a)

            @pl.when(w == NS - 1)
            def _():
                pltpu.sync_copy(src_at(RPT), pay_a)
                pltpu.sync_copy(pay_a, acc.at[idx_v.at[RPT, 0]], add=True)

            plsc.subcore_barrier()
            pltpu.sync_copy(acc.at[pl.ds(w * 624, 624)],
                            out_hbm.at[c, p, pl.ds(w * 624, 624)])

            @pl.when(w == NS - 1)
            def _():
                pltpu.sync_copy(acc.at[pl.ds(N - 16, 16)],
                                out_hbm.at[c, p, pl.ds(N - 16, 16)])

            plsc.subcore_barrier()

    return k(P, G, i0r)


def _node_body(a_ref, s_ref, sb_ref, vx_r, vy_r, vz_r, wx_r, wy_r, wz_r,
               Wvb_ref, Wv_ref,
               dh_ref, dhb_ref, dvx_r, dvy_r, dvz_r, dbx_r, dby_r, dbz_r):
    agg = lambda p: a_ref[p] + a_ref[NPL + p]
    dh = agg(0)
    pv = (agg(1), agg(2), agg(3))
    B = (agg(4), agg(5), agg(6))
    Av = (agg(7), agg(8), agg(9))
    Aw = (agg(10), agg(11), agg(12))
    s = s_ref[...]
    sb = sb_ref[...]
    v = (vx_r[...], vy_r[...], vz_r[...])
    vb = (wx_r[...], wy_r[...], wz_r[...])
    dh_ref[...] = dh
    dhb_ref[...] = v[0] * B[0] + v[1] * B[1] + v[2] * B[2]

    def cross(a, b, d):
        i, j = (d + 1) % 3, (d + 2) % 3
        return a[i] * b[j] - a[j] * b[i]

    Wvb = Wvb_ref[...]
    Wv = Wv_ref[...]
    douts = (dvx_r, dvy_r, dvz_r)
    bouts = (dbx_r, dby_r, dbz_r)
    for d in range(3):
        catb = jnp.concatenate(
            [s * Aw[d], sb * Av[d], cross(v, Av, d), cross(vb, Aw, d)], axis=1)
        bouts[d][...] = jnp.dot(catb, Wvb, preferred_element_type=jnp.float32,
                                precision=_HI)
        catv = jnp.concatenate(
            [s * Av[d], sb * Aw[d], cross(v, Aw, d)], axis=1)
        douts[d][...] = pv[d] + jnp.dot(
            catv, Wv, preferred_element_type=jnp.float32, precision=_HI)


def _node_stage(A, s_j, sbar_j, vx, vy, vz, wx, wy, wz, Wdvbar, Wdv):
    nb = 10
    bn = N // nb
    row = lambda i: (i, 0)
    o = pl.BlockSpec((bn, FEAT), row)
    outs = [jax.ShapeDtypeStruct((N, FEAT), jnp.float32)] * 8
    return pl.pallas_call(
        _node_body,
        grid=(nb,),
        in_specs=[
            pl.BlockSpec((2 * NPL, bn, FEAT), lambda i: (0, i, 0)),
        ] + [pl.BlockSpec((bn, FEAT), row)] * 8 + [
            pl.BlockSpec((H4, FEAT), lambda i: (0, 0)),
            pl.BlockSpec((3 * FEAT, FEAT), lambda i: (0, 0)),
        ],
        out_specs=[o] * 8,
        out_shape=outs,
    )(A, s_j, sbar_j, vx, vy, vz, wx, wy, wz, Wdvbar, Wdv)


def kernel(s_j, sbar_j, v_j, vbar_j, r_ij, nbrs, W1, b1, W2, b2, Wd, bd,
           Wdvbar, Wdv):
    i0 = nbrs[:, 0].astype(jnp.int32)
    i1 = nbrs[:, 1].astype(jnp.int32)
    vx, vy, vz = v_j[:, :, 0], v_j[:, :, 1], v_j[:, :, 2]
    wx, wy, wz = vbar_j[:, :, 0], vbar_j[:, :, 1], vbar_j[:, :, 2]
    rx, ry, rz = (r_ij[:, 0].reshape(1, E), r_ij[:, 1].reshape(1, E),
                  r_ij[:, 2].reshape(1, E))
    Wd21 = jnp.concatenate([Wd, bd.reshape(1, H4)], axis=0)

    table = _build_table(s_j, vx, vy, vz, wx, wy, wz, W1, b1, W2, b2)
    G = _gather_rows(table, i1.reshape(NT, NGC, GCH))
    RB = _rchain(rx, ry, rz).T
    P = _edge_stage(G, RB, Wd21)
    parts = _scatter_planes(P, G, i0.reshape(NROW, 1, RPB))
    A = parts.reshape(NC * NPL, N, FEAT)
    dh, dhbar, dvx, dvy, dvz, dbx, dby, dbz = _node_stage(
        A, s_j, sbar_j, vx, vy, vz, wx, wy, wz, Wdvbar, Wdv)
    dv = jnp.stack([dvx, dvy, dvz], axis=-1)
    dvbar = jnp.stack([dbx, dby, dbz], axis=-1)
    return dh, dhbar, dv, dvbar
